# baseline (speedup 1.0000x reference)
import functools

import jax
import jax.numpy as jnp
from jax import lax
from jax.experimental import pallas as pl
from jax.experimental.pallas import tpu as pltpu
from jax.experimental.pallas import tpu_sc as plsc

B, S = 2, 2048
NT = B * S
H = 2880
E = 32
TOPK = 4
NW = 32
TPW = NT // NW
TBLK = 128
NBLK = NT // TBLK
LANES = 16
NGRP = TPW // LANES


def _tc_body(mask_ref, hs_ref, w_ref, b_ref, hout_ref, st_ref):
    m = mask_ref[...]
    h = hs_ref[...]
    m3 = m * m * m
    hm3 = h * m3
    hout_ref[...] = hm3 * m
    st = lax.dot_general(hm3, w_ref[...], (((1,), (1,)), ((), ())),
                         preferred_element_type=jnp.float32)
    st = st + b_ref[...]
    st_ref[...] = st.T[None]


_tc_call = pl.pallas_call(
    _tc_body,
    grid=(NBLK,),
    in_specs=[
        pl.BlockSpec((TBLK, 1), lambda i: (i, 0)),
        pl.BlockSpec((TBLK, H), lambda i: (i, 0)),
        pl.BlockSpec((E, H), lambda i: (0, 0)),
        pl.BlockSpec((1, E), lambda i: (0, 0)),
    ],
    out_specs=[
        pl.BlockSpec((TBLK, H), lambda i: (i, 0)),
        pl.BlockSpec((1, E, TBLK), lambda i: (i, 0, 0)),
    ],
    out_shape=[
        jax.ShapeDtypeStruct((NT, H), jnp.float32),
        jax.ShapeDtypeStruct((NBLK, E, TBLK), jnp.float32),
    ],
    compiler_params=pltpu.CompilerParams(
        dimension_semantics=("arbitrary",),
    ),
)


@functools.cache
def _make_sc_topk():
    return pl.kernel(
        _sc_topk_body,
        mesh=plsc.VectorSubcoreMesh(core_axis_name="c", subcore_axis_name="s"),
        out_type=[
            jax.ShapeDtypeStruct((NW, TOPK, TPW), jnp.int32),
            jax.ShapeDtypeStruct((NW, TOPK, TPW), jnp.float32),
        ],
        scratch_types=[
            pltpu.VMEM((E, TPW), jnp.float32),
            pltpu.VMEM((TOPK, TPW), jnp.int32),
            pltpu.VMEM((TOPK, TPW), jnp.float32),
        ],
    )


def _sc_topk_body(scores_hbm, iout_hbm, wout_hbm, s_v, i_v, w_v):
    wid = lax.axis_index("s") * 2 + lax.axis_index("c")
    pltpu.sync_copy(scores_hbm.at[wid], s_v)

    def group(g, carry):
        offs = pl.multiple_of(g * LANES, LANES)
        neg = jnp.full((LANES,), -jnp.inf, jnp.float32)
        zero = jnp.zeros((LANES,), jnp.int32)
        b0, b1, b2, b3 = neg, neg, neg, neg
        i0, i1, i2, i3 = zero, zero, zero, zero
        for e in range(E):
            v = s_v[e, pl.ds(offs, LANES)]
            ev = jnp.full((LANES,), e, jnp.int32)
            c0 = v > b0
            c1 = v > b1
            c2 = v > b2
            c3 = v > b3
            b0, b1, b2, b3 = (
                jnp.where(c0, v, b0),
                jnp.where(c0, b0, jnp.where(c1, v, b1)),
                jnp.where(c1, b1, jnp.where(c2, v, b2)),
                jnp.where(c2, b2, jnp.where(c3, v, b3)),
            )
            i0, i1, i2, i3 = (
                jnp.where(c0, ev, i0),
                jnp.where(c0, i0, jnp.where(c1, ev, i1)),
                jnp.where(c1, i1, jnp.where(c2, ev, i2)),
                jnp.where(c2, i2, jnp.where(c3, ev, i3)),
            )
        e0 = jnp.exp(b0 - b0)
        e1 = jnp.exp(b1 - b0)
        e2 = jnp.exp(b2 - b0)
        e3 = jnp.exp(b3 - b0)
        s = e0 + e1 + e2 + e3
        w_v[0, pl.ds(offs, LANES)] = e0 / s
        w_v[1, pl.ds(offs, LANES)] = e1 / s
        w_v[2, pl.ds(offs, LANES)] = e2 / s
        w_v[3, pl.ds(offs, LANES)] = e3 / s
        i_v[0, pl.ds(offs, LANES)] = i0
        i_v[1, pl.ds(offs, LANES)] = i1
        i_v[2, pl.ds(offs, LANES)] = i2
        i_v[3, pl.ds(offs, LANES)] = i3
        return carry

    lax.fori_loop(0, NGRP, group, 0)
    pltpu.sync_copy(i_v, iout_hbm.at[wid])
    pltpu.sync_copy(w_v, wout_hbm.at[wid])


def kernel(input_ids, attention_mask, hidden_states, router_w, router_b):
    hs2d = hidden_states.reshape(NT, H)
    m2d = attention_mask.astype(jnp.float32).reshape(NT, 1)
    w_last = router_w[-1]
    b_last = router_b[-1].reshape(1, E)
    hout, st = _tc_call(m2d, hs2d, w_last, b_last)
    idx, wts = _make_sc_topk()(st)
    h = hout.reshape(B, S, H)
    ei = jnp.transpose(idx, (0, 2, 1)).reshape(B, S, TOPK)
    ew = jnp.transpose(wts, (0, 2, 1)).reshape(B, S, TOPK)
    return h, ei, ew
